# baseline (speedup 1.0000x reference)
import functools

import jax
import jax.numpy as jnp
from jax import lax
from jax.experimental import pallas as pl
from jax.experimental.pallas import tpu as pltpu
from jax.experimental.pallas import tpu_sc as plsc

N_NODES = 1000000
N_RELATIONS = 1000
EMBED_DIM = 64
BATCH = 16384

NC = 2
NS = 16
LANES = 16
NW = NC * NS
BPW = BATCH // NW


def _body(node_hbm, rel_hbm, hidx_hbm, tidx_hbm, ridx_hbm, out_hbm,
          hidx_v, tidx_v, ridx_v, h_v, t_v, r_v, out_v, tr_v,
          sem_h, sem_t, sem_r):
    c = lax.axis_index("c")
    s = lax.axis_index("s")
    wid = s * NC + c
    base = wid * BPW

    pltpu.sync_copy(hidx_hbm.at[pl.ds(base, BPW)], hidx_v)
    pltpu.sync_copy(tidx_hbm.at[pl.ds(base, BPW)], tidx_v)
    pltpu.sync_copy(ridx_hbm.at[pl.ds(base, BPW)], ridx_v)

    ch = pltpu.async_copy(node_hbm.at[hidx_v], h_v, sem_h)
    ct = pltpu.async_copy(node_hbm.at[tidx_v], t_v, sem_t)
    cr = pltpu.async_copy(rel_hbm.at[ridx_v], r_v, sem_r)
    ch.wait()
    ct.wait()
    cr.wait()

    lane = lax.iota(jnp.int32, 16)
    col_base = lane * 17

    def chunk(cix, carry):
        for l in range(LANES):
            b = cix * LANES + l
            acc = None
            for j in range(EMBED_DIM // LANES):
                sl = pl.ds(j * LANES, LANES)
                p = h_v[b, sl] * r_v[b, sl] * t_v[b, sl]
                acc = p if acc is None else acc + p
            tr_v[pl.ds(l * 17, LANES)] = acc
        outs = None
        for j in range(LANES):
            col = plsc.load_gather(tr_v, [col_base + j])
            outs = col if outs is None else outs + col
        out_v[pl.ds(cix * LANES, LANES)] = outs
        return carry

    lax.fori_loop(0, BPW // LANES, chunk, 0)

    pltpu.sync_copy(out_v, out_hbm.at[pl.ds(base, BPW)])


@functools.partial(jax.jit, static_argnames=())
def _run(head_indices, tail_indices, relation_indices,
         node_embedding, relation_embedding):
    mesh = plsc.VectorSubcoreMesh(core_axis_name="c", subcore_axis_name="s",
                                  num_cores=NC, num_subcores=NS)
    k = pl.kernel(
        _body,
        out_type=jax.ShapeDtypeStruct((BATCH,), jnp.float32),
        mesh=mesh,
        compiler_params=pltpu.CompilerParams(needs_layout_passes=False,
                                             use_tc_tiling_on_sc=False),
        scratch_types=[
            pltpu.VMEM((BPW,), jnp.int32),
            pltpu.VMEM((BPW,), jnp.int32),
            pltpu.VMEM((BPW,), jnp.int32),
            pltpu.VMEM((BPW, EMBED_DIM), jnp.float32),
            pltpu.VMEM((BPW, EMBED_DIM), jnp.float32),
            pltpu.VMEM((BPW, EMBED_DIM), jnp.float32),
            pltpu.VMEM((BPW,), jnp.float32),
            pltpu.VMEM((LANES * 17,), jnp.float32),
            pltpu.SemaphoreType.DMA,
            pltpu.SemaphoreType.DMA,
            pltpu.SemaphoreType.DMA,
        ],
    )
    return k(node_embedding, relation_embedding,
             head_indices.astype(jnp.int32),
             tail_indices.astype(jnp.int32),
             relation_indices.astype(jnp.int32))


def kernel(head_indices, tail_indices, relation_indices,
           node_embedding, relation_embedding):
    return _run(head_indices, tail_indices, relation_indices,
                node_embedding, relation_embedding)


# speedup vs baseline: 1.6498x; 1.6498x over previous
import functools

import jax
import jax.numpy as jnp
from jax import lax
from jax.experimental import pallas as pl
from jax.experimental.pallas import tpu as pltpu
from jax.experimental.pallas import tpu_sc as plsc

N_NODES = 1000000
N_RELATIONS = 1000
EMBED_DIM = 64
BATCH = 16384

NC = 2
NS = 16
LANES = 16
NW = NC * NS
BPW = BATCH // NW
NCHUNK = 4
CK = BPW // NCHUNK
UNROLL = 8


def _body(node_hbm, rel_hbm, hidx_hbm, tidx_hbm, ridx_hbm, out_hbm,
          h_vi, t_vi, r_vi, h_v, t_v, r_v, out_v, tr_v,
          sems):
    c = lax.axis_index("c")
    s = lax.axis_index("s")
    wid = s * NC + c
    base = wid * BPW

    def load_indices(cix, par):
        off = base + cix * CK
        pltpu.sync_copy(hidx_hbm.at[pl.ds(off, CK)], h_vi.at[par])
        pltpu.sync_copy(tidx_hbm.at[pl.ds(off, CK)], t_vi.at[par])
        pltpu.sync_copy(ridx_hbm.at[pl.ds(off, CK)], r_vi.at[par])


    def issue_gathers(par):
        def step(i, carry):
            hvec = h_vi[par, pl.ds(i * LANES, LANES)]
            tvec = t_vi[par, pl.ds(i * LANES, LANES)]
            rvec = r_vi[par, pl.ds(i * LANES, LANES)]
            for u in range(LANES):
                k = i * LANES + u
                pltpu.async_copy(node_hbm.at[pl.ds(hvec[u], 1)],
                                 h_v.at[par, pl.ds(k, 1)], sems.at[par])
                pltpu.async_copy(node_hbm.at[pl.ds(tvec[u], 1)],
                                 t_v.at[par, pl.ds(k, 1)], sems.at[par])
                pltpu.async_copy(rel_hbm.at[pl.ds(rvec[u], 1)],
                                 r_v.at[par, pl.ds(k, 1)], sems.at[par])
            return carry
        lax.fori_loop(0, CK // LANES, step, 0)

    def drain_gathers(par):
        def step(i, carry):
            for u in range(UNROLL):
                k = i * UNROLL + u
                pltpu.make_async_copy(node_hbm.at[pl.ds(0, 1)],
                                      h_v.at[par, pl.ds(k, 1)],
                                      sems.at[par]).wait()
                pltpu.make_async_copy(node_hbm.at[pl.ds(0, 1)],
                                      t_v.at[par, pl.ds(k, 1)],
                                      sems.at[par]).wait()
                pltpu.make_async_copy(rel_hbm.at[pl.ds(0, 1)],
                                      r_v.at[par, pl.ds(k, 1)],
                                      sems.at[par]).wait()
            return carry
        lax.fori_loop(0, CK // UNROLL, step, 0)

    lane = lax.iota(jnp.int32, 16)
    col_base = lane * 17

    def compute(cix, par):
        def group(g, carry):
            for l in range(LANES):
                b = g * LANES + l
                acc = None
                for j in range(EMBED_DIM // LANES):
                    sl = pl.ds(j * LANES, LANES)
                    p = h_v[par, b, sl] * r_v[par, b, sl] * t_v[par, b, sl]
                    acc = p if acc is None else acc + p
                tr_v[pl.ds(l * 17, LANES)] = acc
            outs = None
            for j in range(LANES):
                col = plsc.load_gather(tr_v, [col_base + j])
                outs = col if outs is None else outs + col
            out_v[pl.ds(cix * CK + g * LANES, LANES)] = outs
            return carry
        lax.fori_loop(0, CK // LANES, group, 0)

    load_indices(0, 0)
    issue_gathers(0)
    for cix in range(NCHUNK):
        par = cix % 2
        if cix + 1 < NCHUNK:
            load_indices(cix + 1, 1 - par)
            issue_gathers(1 - par)
        drain_gathers(par)
        compute(cix, par)

    pltpu.sync_copy(out_v, out_hbm.at[pl.ds(base, BPW)])


@functools.partial(jax.jit, static_argnames=())
def _run(head_indices, tail_indices, relation_indices,
         node_embedding, relation_embedding):
    mesh = plsc.VectorSubcoreMesh(core_axis_name="c", subcore_axis_name="s",
                                  num_cores=NC, num_subcores=NS)
    k = pl.kernel(
        _body,
        out_type=jax.ShapeDtypeStruct((BATCH,), jnp.float32),
        mesh=mesh,
        compiler_params=pltpu.CompilerParams(needs_layout_passes=False,
                                             use_tc_tiling_on_sc=True),
        scratch_types=[
            pltpu.VMEM((2, CK), jnp.int32),
            pltpu.VMEM((2, CK), jnp.int32),
            pltpu.VMEM((2, CK), jnp.int32),
            pltpu.VMEM((2, CK, EMBED_DIM), jnp.float32),
            pltpu.VMEM((2, CK, EMBED_DIM), jnp.float32),
            pltpu.VMEM((2, CK, EMBED_DIM), jnp.float32),
            pltpu.VMEM((BPW,), jnp.float32),
            pltpu.VMEM((LANES * 17,), jnp.float32),
            pltpu.SemaphoreType.DMA((2,)),
        ],
    )
    return k(node_embedding, relation_embedding,
             head_indices.astype(jnp.int32),
             tail_indices.astype(jnp.int32),
             relation_indices.astype(jnp.int32))


def kernel(head_indices, tail_indices, relation_indices,
           node_embedding, relation_embedding):
    return _run(head_indices, tail_indices, relation_indices,
                node_embedding, relation_embedding)
